# baseline (speedup 1.0000x reference)
import jax
import jax.numpy as jnp
from jax import lax
from jax.experimental import pallas as pl
from jax.experimental.pallas import tpu as pltpu
from jax.experimental.pallas import tpu_sc as plsc

BATCH = 4096
SEQ_LEN = 200
DEPTH = 32
NW = 32
BROWS_PER_W = BATCH // NW
GROUP_ROWS = 4
N_GROUPS = BROWS_PER_W // GROUP_ROWS


def _sc_body(seq_hbm, pos_hbm, table_hbm, out_hbm,
             idx_v, rows0, rows1, pos_v, gsem0, gsem1, osem0, osem1):
    wid = lax.axis_index("s") * 2 + lax.axis_index("c")
    brow0 = wid * BROWS_PER_W
    pltpu.sync_copy(seq_hbm.at[pl.ds(brow0, BROWS_PER_W)], idx_v)
    pltpu.sync_copy(pos_hbm, pos_v)

    def gather(g, rows_ref, sem):
        for j in range(GROUP_ROWS):
            pltpu.async_copy(table_hbm.at[idx_v.at[g * GROUP_ROWS + j]],
                             rows_ref.at[j], sem)

    def gather_wait(rows_ref, sem):
        for j in range(GROUP_ROWS):
            pltpu.make_async_copy(table_hbm.at[idx_v.at[0]],
                                  rows_ref.at[j], sem).wait()

    def store(g, rows_ref, sem):
        pltpu.async_copy(
            rows_ref, out_hbm.at[pl.ds(brow0 + g * GROUP_ROWS, GROUP_ROWS)],
            sem)

    def store_wait(rows_ref, sem):
        pltpu.make_async_copy(rows_ref,
                              out_hbm.at[pl.ds(brow0, GROUP_ROWS)], sem).wait()

    def add_pos(rows_ref):
        @pl.loop(0, SEQ_LEN, unroll=2)
        def _(l):
            p0 = pos_v[l, pl.ds(0, 16)]
            p1 = pos_v[l, pl.ds(16, 16)]
            for j in range(GROUP_ROWS):
                plsc.addupdate(rows_ref.at[j, l, pl.ds(0, 16)], p0)
                plsc.addupdate(rows_ref.at[j, l, pl.ds(16, 16)], p1)

    gather(0, rows0, gsem0)

    def pipe_body(i, carry):
        g0 = 2 * i

        @pl.when(g0 > 0)
        def _():
            store_wait(rows1, osem1)
        gather(g0 + 1, rows1, gsem1)

        gather_wait(rows0, gsem0)
        add_pos(rows0)
        store(g0, rows0, osem0)

        @pl.when(g0 + 2 < N_GROUPS)
        def _():
            store_wait(rows0, osem0)
            gather(g0 + 2, rows0, gsem0)

        gather_wait(rows1, gsem1)
        add_pos(rows1)
        store(g0 + 1, rows1, osem1)
        return carry

    lax.fori_loop(0, N_GROUPS // 2, pipe_body, 0)
    store_wait(rows0, osem0)
    store_wait(rows1, osem1)


def kernel(seq, token_table, pos_table):
    mesh = plsc.VectorSubcoreMesh(core_axis_name="c", subcore_axis_name="s")
    out = pl.kernel(
        _sc_body,
        out_type=jax.ShapeDtypeStruct((BATCH, SEQ_LEN, DEPTH), jnp.float32),
        mesh=mesh,
        compiler_params=pltpu.CompilerParams(use_tc_tiling_on_sc=False),
        scratch_types=[
            pltpu.VMEM((BROWS_PER_W, SEQ_LEN), jnp.int32),
            pltpu.VMEM((GROUP_ROWS, SEQ_LEN, DEPTH), jnp.float32),
            pltpu.VMEM((GROUP_ROWS, SEQ_LEN, DEPTH), jnp.float32),
            pltpu.VMEM((SEQ_LEN, DEPTH), jnp.float32),
            pltpu.SemaphoreType.DMA,
            pltpu.SemaphoreType.DMA,
            pltpu.SemaphoreType.DMA,
            pltpu.SemaphoreType.DMA,
        ],
    )(seq.astype(jnp.int32), pos_table, token_table)
    return out
